# Initial kernel scaffold; baseline (speedup 1.0000x reference)
import jax
import jax.numpy as jnp
from jax.experimental import pallas as pl


def kernel(x, W1, b1, W2, b2, W3, b3, codebook, D1, d1, D2, d2, D3, d3, D4, d4):
    raise NotImplementedError("write your pallas kernel here")


# speedup vs baseline: 8.1488x; 8.1488x over previous
import functools

import jax
import jax.numpy as jnp
from jax.experimental import pallas as pl
from jax.experimental.pallas import tpu as pltpu

B, INPUT_DIMS, CODE_DIM, K, NUM_ACTIONS = 8192, 768, 64, 512, 768
BM = 1024


def _block_kernel(x_ref, W1_ref, b1_ref, W2_ref, b2_ref, W3_ref, b3_ref,
                  cb_ref, cbT_ref, D1_ref, d1_ref, D2_ref, d2_ref, D3_ref,
                  d3_ref, D4_ref, d4_ref, out_ref):
    f32 = jnp.float32
    dot = functools.partial(jnp.dot, preferred_element_type=f32)

    x = x_ref[:]
    h = jnp.maximum(dot(x, W1_ref[:]) + b1_ref[:], 0.0)
    h = jnp.maximum(dot(h, W2_ref[:]) + b2_ref[:], 0.0)
    hdot = functools.partial(jnp.dot, preferred_element_type=f32,
                             precision=jax.lax.Precision.HIGHEST)
    z_e = hdot(h, W3_ref[:]) + b3_ref[:]

    cbT = cbT_ref[:]
    cnorm = jnp.sum(cbT * cbT, axis=0)[None, :]
    d2 = cnorm - 2.0 * hdot(z_e, cbT)

    dmin = jnp.min(d2, axis=1, keepdims=True)
    lane = jax.lax.broadcasted_iota(jnp.int32, d2.shape, 1)
    masked = jnp.where(d2 <= dmin, lane, K)
    idx = jnp.min(masked, axis=1, keepdims=True)

    onehot = (lane == idx).astype(f32)
    z_q = dot(onehot, cb_ref[:])

    h = jnp.maximum(dot(z_q, D1_ref[:]) + d1_ref[:], 0.0)
    h = jnp.maximum(dot(h, D2_ref[:]) + d2_ref[:], 0.0)
    h = jnp.maximum(dot(h, D3_ref[:]) + d3_ref[:], 0.0)
    out_ref[:] = dot(h, D4_ref[:]) + d4_ref[:]


def kernel(x, W1, b1, W2, b2, W3, b3, codebook, D1, d1, D2, d2, D3, d3, D4, d4):
    b1r, b2r, b3r = b1[None, :], b2[None, :], b3[None, :]
    d1r, d2r, d3r, d4r = d1[None, :], d2[None, :], d3[None, :], d4[None, :]
    cbT = codebook.T

    grid = (B // BM,)
    row_spec = lambda n: pl.BlockSpec((BM, n), lambda i: (i, 0))
    full = lambda a: pl.BlockSpec(a.shape, lambda i: tuple(0 for _ in a.shape))

    out = pl.pallas_call(
        _block_kernel,
        grid=grid,
        in_specs=[
            row_spec(INPUT_DIMS),
            full(W1), full(b1r), full(W2), full(b2r), full(W3), full(b3r),
            full(codebook), full(cbT),
            full(D1), full(d1r), full(D2), full(d2r), full(D3), full(d3r),
            full(D4), full(d4r),
        ],
        out_specs=row_spec(NUM_ACTIONS),
        out_shape=jax.ShapeDtypeStruct((B, NUM_ACTIONS), jnp.float32),
        compiler_params=pltpu.CompilerParams(
            dimension_semantics=("arbitrary",),
        ),
    )(x, W1, b1r, W2, b2r, W3, b3r, codebook, cbT,
      D1, d1r, D2, d2r, D3, d3r, D4, d4r)
    return out


# speedup vs baseline: 10.2528x; 1.2582x over previous
import functools

import jax
import jax.numpy as jnp
from jax.experimental import pallas as pl
from jax.experimental.pallas import tpu as pltpu

B, INPUT_DIMS, CODE_DIM, K, NUM_ACTIONS = 8192, 768, 64, 512, 768
BM = 1024


def _block_kernel(x_ref, W1_ref, b1_ref, W2_ref, b2_ref, W3_ref, b3_ref,
                  cb_ref, cbT_ref, D1_ref, d1_ref, D2_ref, d2_ref, D3_ref,
                  d3_ref, D4_ref, d4_ref, out_ref):
    f32 = jnp.float32
    dot = functools.partial(jnp.dot, preferred_element_type=f32)

    x = x_ref[:]
    h = jnp.maximum(dot(x, W1_ref[:]) + b1_ref[:], 0.0)
    h = jnp.maximum(dot(h, W2_ref[:]) + b2_ref[:], 0.0)
    hdot = functools.partial(jnp.dot, preferred_element_type=f32,
                             precision=jax.lax.Precision.HIGHEST)
    z_e = dot(h, W3_ref[:]) + b3_ref[:]

    cbT = cbT_ref[:]
    cnorm = jnp.sum(cbT * cbT, axis=0)[None, :]
    d2 = cnorm - 2.0 * hdot(z_e, cbT)

    dmin = jnp.min(d2, axis=1, keepdims=True)
    lane = jax.lax.broadcasted_iota(jnp.int32, d2.shape, 1)
    masked = jnp.where(d2 <= dmin, lane, K)
    idx = jnp.min(masked, axis=1, keepdims=True)

    onehot = (lane == idx).astype(f32)
    z_q = dot(onehot, cb_ref[:])

    h = jnp.maximum(dot(z_q, D1_ref[:]) + d1_ref[:], 0.0)
    h = jnp.maximum(dot(h, D2_ref[:]) + d2_ref[:], 0.0)
    h = jnp.maximum(dot(h, D3_ref[:]) + d3_ref[:], 0.0)
    out_ref[:] = dot(h, D4_ref[:]) + d4_ref[:]


def kernel(x, W1, b1, W2, b2, W3, b3, codebook, D1, d1, D2, d2, D3, d3, D4, d4):
    b1r, b2r, b3r = b1[None, :], b2[None, :], b3[None, :]
    d1r, d2r, d3r, d4r = d1[None, :], d2[None, :], d3[None, :], d4[None, :]
    cbT = codebook.T

    grid = (B // BM,)
    row_spec = lambda n: pl.BlockSpec((BM, n), lambda i: (i, 0))
    full = lambda a: pl.BlockSpec(a.shape, lambda i: tuple(0 for _ in a.shape))

    out = pl.pallas_call(
        _block_kernel,
        grid=grid,
        in_specs=[
            row_spec(INPUT_DIMS),
            full(W1), full(b1r), full(W2), full(b2r), full(W3), full(b3r),
            full(codebook), full(cbT),
            full(D1), full(d1r), full(D2), full(d2r), full(D3), full(d3r),
            full(D4), full(d4r),
        ],
        out_specs=row_spec(NUM_ACTIONS),
        out_shape=jax.ShapeDtypeStruct((B, NUM_ACTIONS), jnp.float32),
        compiler_params=pltpu.CompilerParams(
            dimension_semantics=("arbitrary",),
        ),
    )(x, W1, b1r, W2, b2r, W3, b3r, codebook, cbT,
      D1, d1r, D2, d2r, D3, d3r, D4, d4r)
    return out


# speedup vs baseline: 11.9857x; 1.1690x over previous
import functools

import jax
import jax.numpy as jnp
from jax.experimental import pallas as pl
from jax.experimental.pallas import tpu as pltpu

B, INPUT_DIMS, CODE_DIM, K, NUM_ACTIONS = 8192, 768, 64, 512, 768
BM = 1024


def _block_kernel(x_ref, W1_ref, b1_ref, W2_ref, b2_ref, W3_ref, b3_ref,
                  cb_ref, cbT_ref, cbTlo_ref, D1_ref, d1_ref, D2_ref, d2_ref,
                  D3_ref, d3_ref, D4_ref, d4_ref, out_ref):
    f32 = jnp.float32
    dot = functools.partial(jnp.dot, preferred_element_type=f32)

    x = x_ref[:]
    h = jnp.maximum(dot(x, W1_ref[:]) + b1_ref[:], 0.0)
    h = jnp.maximum(dot(h, W2_ref[:]) + b2_ref[:], 0.0)
    z_e = dot(h, W3_ref[:]) + b3_ref[:]

    cbT = cbT_ref[:]
    cnorm = jnp.sum(cbT * cbT, axis=0)[None, :]
    z_lo = z_e - z_e.astype(jnp.bfloat16).astype(f32)
    scores = dot(z_e, cbT) + dot(z_lo, cbTlo_ref[:])
    d2 = cnorm - 2.0 * scores

    dmin = jnp.min(d2, axis=1, keepdims=True)
    lane = jax.lax.broadcasted_iota(jnp.int32, d2.shape, 1)
    masked = jnp.where(d2 <= dmin, lane, K)
    idx = jnp.min(masked, axis=1, keepdims=True)

    onehot = (lane == idx).astype(f32)
    z_q = dot(onehot, cb_ref[:])

    h = jnp.maximum(dot(z_q, D1_ref[:]) + d1_ref[:], 0.0)
    h = jnp.maximum(dot(h, D2_ref[:]) + d2_ref[:], 0.0)
    h = jnp.maximum(dot(h, D3_ref[:]) + d3_ref[:], 0.0)
    out_ref[:] = dot(h, D4_ref[:]) + d4_ref[:]


def kernel(x, W1, b1, W2, b2, W3, b3, codebook, D1, d1, D2, d2, D3, d3, D4, d4):
    b1r, b2r, b3r = b1[None, :], b2[None, :], b3[None, :]
    d1r, d2r, d3r, d4r = d1[None, :], d2[None, :], d3[None, :], d4[None, :]
    cbT = codebook.T
    cbTlo = cbT - cbT.astype(jnp.bfloat16).astype(jnp.float32)

    grid = (B // BM,)
    row_spec = lambda n: pl.BlockSpec((BM, n), lambda i: (i, 0))
    full = lambda a: pl.BlockSpec(a.shape, lambda i: tuple(0 for _ in a.shape))

    out = pl.pallas_call(
        _block_kernel,
        grid=grid,
        in_specs=[
            row_spec(INPUT_DIMS),
            full(W1), full(b1r), full(W2), full(b2r), full(W3), full(b3r),
            full(codebook), full(cbT), full(cbTlo),
            full(D1), full(d1r), full(D2), full(d2r), full(D3), full(d3r),
            full(D4), full(d4r),
        ],
        out_specs=row_spec(NUM_ACTIONS),
        out_shape=jax.ShapeDtypeStruct((B, NUM_ACTIONS), jnp.float32),
        compiler_params=pltpu.CompilerParams(
            dimension_semantics=("arbitrary",),
        ),
    )(x, W1, b1r, W2, b2r, W3, b3r, codebook, cbT, cbTlo,
      D1, d1r, D2, d2r, D3, d3r, D4, d4r)
    return out
